# baseline (speedup 1.0000x reference)
import jax
import jax.numpy as jnp
from jax import lax
from jax.experimental import pallas as pl
from jax.experimental.pallas import tpu as pltpu
from jax.experimental.pallas import tpu_sc as plsc

_N = 1024
_HW = 784
_NC = 81
_ROW = _HW * _NC
_NH = _N // 2

_NCORES = 2
_NSUB = 16
_NW = _NCORES * _NSUB
_RPH = _NH // _NW
_RPW = 2 * _RPH
_CHUNK = 112
_CPR = _HW // _CHUNK
_ROWS = _RPW * _CPR

_BB = 128
_NBLK = _N // _BB


def _sc_gather_body(pred_a, pred_b, ids_hbm, out_hbm, ids_v, idx_v, buf_v,
                    sem):
    c = lax.axis_index("c")
    s = lax.axis_index("s")
    wid = s * _NCORES + c
    lanes = lax.iota(jnp.int32, 16)

    pltpu.sync_copy(ids_hbm.at[pl.ds(wid * _RPH, _RPH)],
                    ids_v.at[pl.ds(0, _RPH)])
    pltpu.sync_copy(ids_hbm.at[pl.ds(_NH + wid * _RPH, _RPH)],
                    ids_v.at[pl.ds(_RPH, _RPH)])

    def half(table, slot):
        def per_roi(jj, carry):
            j = slot * _RPH + jj
            grp = ids_v[pl.ds(slot * _RPH, 16)]
            c_id = jnp.sum(jnp.where(lanes == jj, grp, 0))
            rbase = (wid * _RPH + jj) * _ROW
            base = rbase + c_id + lanes * _NC

            def chunk(a, _):
                vec = base + a * (_NC * 16)
                idx_v[j * _CPR + a // 7, pl.ds((a % 7) * 16, 16)] = vec
                return _

            lax.fori_loop(0, 49, chunk, 0, unroll=False)

            def f(r, _):
                row = j * _CPR + r
                pltpu.async_copy(
                    table.at[idx_v.at[row]],
                    buf_v.at[pl.ds(row * _CHUNK, _CHUNK)], sem)
                return _

            lax.fori_loop(0, _CPR, f, 0, unroll=False)

            def d(r, _):
                row = j * _CPR + r
                pltpu.make_async_copy(
                    table.at[idx_v.at[row]],
                    buf_v.at[pl.ds(row * _CHUNK, _CHUNK)], sem).wait()
                return _

            lax.fori_loop(0, _CPR, d, 0, unroll=False)
            return carry

        lax.fori_loop(0, _RPH, per_roi, 0, unroll=False)
        pltpu.sync_copy(
            buf_v.at[pl.ds(slot * _RPH * _HW, _RPH * _HW)],
            out_hbm.at[pl.ds(slot * _NH * _HW + wid * _RPH * _HW,
                             _RPH * _HW)])

    half(pred_a, 0)
    half(pred_b, 1)


def _bce_body(ids_ref, t_ref, yp_ref, out_ref):
    i = pl.program_id(0)
    ids = ids_ref[0, 0, :]
    t = t_ref[...]
    yp = yp_ref[...]

    eps = jnp.float32(1e-7)
    p = jnp.clip(yp, eps, 1.0 - eps)
    bce = -(t * jnp.log(p) + (1.0 - t) * jnp.log(1.0 - p))
    valid = (ids > 0).astype(jnp.float32)
    bsum = jnp.sum(bce * valid[:, None])
    bcnt = jnp.sum(valid)

    @pl.when(i == 0)
    def _init():
        out_ref[0, 0] = 0.0
        out_ref[0, 1] = 0.0

    out_ref[0, 0] += bsum
    out_ref[0, 1] += bcnt

    @pl.when(i == _NBLK - 1)
    def _fini():
        total = out_ref[0, 0]
        cnt = out_ref[0, 1]
        denom = jnp.maximum(cnt, 1.0) * jnp.float32(_HW)
        out_ref[0, 0] = jnp.where(cnt > 0, total / denom, jnp.float32(0.0))


@jax.jit
def kernel(true_masks, target_class_ids, pred_masks):
    ids = target_class_ids.reshape(_N).astype(jnp.int32)
    pred_a = pred_masks[:2].reshape(_NH * _ROW)
    pred_b = pred_masks[2:].reshape(_NH * _ROW)
    t = true_masks.reshape(_N, _HW)

    gather = pl.kernel(
        _sc_gather_body,
        out_type=jax.ShapeDtypeStruct((_N * _HW,), jnp.float32),
        mesh=plsc.VectorSubcoreMesh(
            core_axis_name="c", subcore_axis_name="s",
            num_cores=_NCORES, num_subcores=_NSUB),
        scratch_types=[
            pltpu.VMEM((_RPW,), jnp.int32),
            pltpu.VMEM((_ROWS, _CHUNK), jnp.int32),
            pltpu.VMEM((_ROWS * _CHUNK,), jnp.float32),
            pltpu.SemaphoreType.DMA,
        ],
        compiler_params=pltpu.CompilerParams(needs_layout_passes=False),
    )
    yp = gather(pred_a, pred_b, ids).reshape(_N, _HW)

    out = pl.pallas_call(
        _bce_body,
        grid=(_NBLK,),
        in_specs=[
            pl.BlockSpec((1, 1, _BB), lambda i: (i, 0, 0)),
            pl.BlockSpec((_BB, _HW), lambda i: (i, 0)),
            pl.BlockSpec((_BB, _HW), lambda i: (i, 0)),
        ],
        out_specs=pl.BlockSpec(
            (1, 2), lambda i: (0, 0), memory_space=pltpu.SMEM
        ),
        out_shape=jax.ShapeDtypeStruct((1, 2), jnp.float32),
    )(ids.reshape(_NBLK, 1, _BB), t, yp)
    return out[0, 0]


# speedup vs baseline: 1.6740x; 1.6740x over previous
import jax
import jax.numpy as jnp
from jax import lax
from jax.experimental import pallas as pl
from jax.experimental.pallas import tpu as pltpu
from jax.experimental.pallas import tpu_sc as plsc

_N = 1024
_HW = 784
_NC = 81
_ROW = _HW * _NC

_NCORES = 2
_NSUB = 16
_NW = _NCORES * _NSUB
_RPW = _N // _NW
_CHUNK = 112
_CPR = _HW // _CHUNK
_ROWS = _RPW * _CPR

_BB = 128
_NBLK = _N // _BB


def _sc_gather_body(pred_hbm, ids_hbm, out_hbm, ids_v, idx_v, buf_v, sem):
    c = lax.axis_index("c")
    s = lax.axis_index("s")
    wid = s * _NCORES + c
    pltpu.sync_copy(ids_hbm.at[pl.ds(wid * _RPW, _RPW)], ids_v)
    lanes = lax.iota(jnp.int32, 16)

    def fire(j, r):
        row = j * _CPR + r
        pltpu.async_copy(
            pred_hbm.at[idx_v.at[row]],
            buf_v.at[pl.ds(row * _CHUNK, _CHUNK)], sem)

    def drain(j, r):
        row = j * _CPR + r
        pltpu.make_async_copy(
            pred_hbm.at[idx_v.at[row]],
            buf_v.at[pl.ds(row * _CHUNK, _CHUNK)], sem).wait()

    def per_roi(j, carry):
        grp = ids_v[pl.ds((j // 16) * 16, 16)]
        c_id = jnp.sum(jnp.where(lanes == (j % 16), grp, 0))
        rbase = (wid * _RPW + j) * _ROW
        base = rbase + c_id + lanes * _NC

        def chunk(a, _):
            vec = base + a * (_NC * 16)
            idx_v[j * _CPR + a // 7, pl.ds((a % 7) * 16, 16)] = vec
            return _

        lax.fori_loop(0, 49, chunk, 0, unroll=False)

        def f(r, _):
            fire(j, r)
            return _

        lax.fori_loop(0, _CPR, f, 0, unroll=False)

        @pl.when(j > 0)
        def _():
            def d(r, _):
                drain(j - 1, r)
                return _

            lax.fori_loop(0, _CPR, d, 0, unroll=False)
        return carry

    lax.fori_loop(0, _RPW, per_roi, 0, unroll=False)

    def dlast(r, _):
        drain(_RPW - 1, r)
        return _

    lax.fori_loop(0, _CPR, dlast, 0, unroll=False)
    pltpu.sync_copy(
        buf_v, out_hbm.at[pl.ds(wid * _ROWS * _CHUNK, _ROWS * _CHUNK)])


def _bce_body(ids_ref, t_ref, yp_ref, out_ref):
    i = pl.program_id(0)
    ids = ids_ref[0, 0, :]
    t = t_ref[...]
    yp = yp_ref[...]

    eps = jnp.float32(1e-7)
    p = jnp.clip(yp, eps, 1.0 - eps)
    bce = -(t * jnp.log(p) + (1.0 - t) * jnp.log(1.0 - p))
    valid = (ids > 0).astype(jnp.float32)
    bsum = jnp.sum(bce * valid[:, None])
    bcnt = jnp.sum(valid)

    @pl.when(i == 0)
    def _init():
        out_ref[0, 0] = 0.0
        out_ref[0, 1] = 0.0

    out_ref[0, 0] += bsum
    out_ref[0, 1] += bcnt

    @pl.when(i == _NBLK - 1)
    def _fini():
        total = out_ref[0, 0]
        cnt = out_ref[0, 1]
        denom = jnp.maximum(cnt, 1.0) * jnp.float32(_HW)
        out_ref[0, 0] = jnp.where(cnt > 0, total / denom, jnp.float32(0.0))


@jax.jit
def kernel(true_masks, target_class_ids, pred_masks):
    ids = target_class_ids.reshape(_N).astype(jnp.int32)
    pred_flat = pred_masks.reshape(_N * _ROW)
    t = true_masks.reshape(_N, _HW)

    gather = pl.kernel(
        _sc_gather_body,
        out_type=jax.ShapeDtypeStruct((_N * _HW,), jnp.float32),
        mesh=plsc.VectorSubcoreMesh(
            core_axis_name="c", subcore_axis_name="s",
            num_cores=_NCORES, num_subcores=_NSUB),
        scratch_types=[
            pltpu.VMEM((_RPW,), jnp.int32),
            pltpu.VMEM((_ROWS, _CHUNK), jnp.int32),
            pltpu.VMEM((_ROWS * _CHUNK,), jnp.float32),
            pltpu.SemaphoreType.DMA,
        ],
        compiler_params=pltpu.CompilerParams(needs_layout_passes=False),
    )
    yp = gather(pred_flat, ids).reshape(_N, _HW)

    out = pl.pallas_call(
        _bce_body,
        grid=(_NBLK,),
        in_specs=[
            pl.BlockSpec((1, 1, _BB), lambda i: (i, 0, 0)),
            pl.BlockSpec((_BB, _HW), lambda i: (i, 0)),
            pl.BlockSpec((_BB, _HW), lambda i: (i, 0)),
        ],
        out_specs=pl.BlockSpec(
            (1, 2), lambda i: (0, 0), memory_space=pltpu.SMEM
        ),
        out_shape=jax.ShapeDtypeStruct((1, 2), jnp.float32),
    )(ids.reshape(_NBLK, 1, _BB), t, yp)
    return out[0, 0]


# speedup vs baseline: 1.6762x; 1.0013x over previous
import jax
import jax.numpy as jnp
from jax import lax
from jax.experimental import pallas as pl
from jax.experimental.pallas import tpu as pltpu
from jax.experimental.pallas import tpu_sc as plsc

_N = 1024
_HW = 784
_NC = 81
_ROW = _HW * _NC

_NCORES = 2
_NSUB = 16
_NW = _NCORES * _NSUB
_RPW = _N // _NW
_CHUNK = 112
_CPR = _HW // _CHUNK
_ROWS = _RPW * _CPR

_BB = 128
_NBLK = _N // _BB


def _sc_gather_body(pred_hbm, ids_hbm, out_hbm, ids_v, idx_v, buf_v, sem):
    c = lax.axis_index("c")
    s = lax.axis_index("s")
    wid = s * _NCORES + c
    pltpu.sync_copy(ids_hbm.at[pl.ds(wid * _RPW, _RPW)], ids_v)
    lanes = lax.iota(jnp.int32, 16)

    def fire(j, r):
        row = j * _CPR + r
        pltpu.async_copy(
            pred_hbm.at[idx_v.at[row]],
            buf_v.at[pl.ds(row * _CHUNK, _CHUNK)], sem)

    def drain(j, r):
        row = j * _CPR + r
        pltpu.make_async_copy(
            pred_hbm.at[idx_v.at[row]],
            buf_v.at[pl.ds(row * _CHUNK, _CHUNK)], sem).wait()

    def per_roi(j, carry):
        grp = ids_v[pl.ds((j // 16) * 16, 16)]
        c_id = jnp.sum(jnp.where(lanes == (j % 16), grp, 0))
        rbase = (wid * _RPW + j) * _ROW
        base = rbase + c_id + lanes * _NC

        def chunk(a, _):
            vec = base + a * (_NC * 16)
            idx_v[j * _CPR + a // 7, pl.ds((a % 7) * 16, 16)] = vec
            return _

        lax.fori_loop(0, 49, chunk, 0, unroll=False)

        def f(r, _):
            fire(j, r)
            return _

        lax.fori_loop(0, _CPR, f, 0, unroll=False)

        @pl.when(j > 0)
        def _():
            def d(r, _):
                drain(j - 1, r)
                return _

            lax.fori_loop(0, _CPR, d, 0, unroll=False)
        return carry

    lax.fori_loop(0, _RPW, per_roi, 0, unroll=False)

    def dlast(r, _):
        drain(_RPW - 1, r)
        return _

    lax.fori_loop(0, _CPR, dlast, 0, unroll=False)
    pltpu.sync_copy(
        buf_v, out_hbm.at[pl.ds(wid * _ROWS * _CHUNK, _ROWS * _CHUNK)])


_FB = _N * _HW // _NBLK


def _bce_body(t_ref, yp_ref, m_ref, out_ref):
    i = pl.program_id(0)
    t = t_ref[...]
    yp = yp_ref[...]
    m = m_ref[...]

    eps = jnp.float32(1e-7)
    p = jnp.clip(yp, eps, 1.0 - eps)
    bce = -(t * jnp.log(p) + (1.0 - t) * jnp.log(1.0 - p))
    bsum = jnp.sum(bce * m)
    bcnt = jnp.sum(m) / jnp.float32(_HW)

    @pl.when(i == 0)
    def _init():
        out_ref[0, 0] = 0.0
        out_ref[0, 1] = 0.0

    out_ref[0, 0] += bsum
    out_ref[0, 1] += bcnt

    @pl.when(i == _NBLK - 1)
    def _fini():
        total = out_ref[0, 0]
        cnt = out_ref[0, 1]
        denom = jnp.maximum(cnt, 1.0) * jnp.float32(_HW)
        out_ref[0, 0] = jnp.where(cnt > 0, total / denom, jnp.float32(0.0))


@jax.jit
def kernel(true_masks, target_class_ids, pred_masks):
    ids = target_class_ids.reshape(_N).astype(jnp.int32)
    pred_flat = pred_masks.reshape(_N * _ROW)
    t = true_masks.reshape(_N, _HW)

    gather = pl.kernel(
        _sc_gather_body,
        out_type=jax.ShapeDtypeStruct((_N * _HW,), jnp.float32),
        mesh=plsc.VectorSubcoreMesh(
            core_axis_name="c", subcore_axis_name="s",
            num_cores=_NCORES, num_subcores=_NSUB),
        scratch_types=[
            pltpu.VMEM((_RPW,), jnp.int32),
            pltpu.VMEM((_ROWS, _CHUNK), jnp.int32),
            pltpu.VMEM((_ROWS * _CHUNK,), jnp.float32),
            pltpu.SemaphoreType.DMA,
        ],
        compiler_params=pltpu.CompilerParams(needs_layout_passes=False),
    )
    yp = gather(pred_flat, ids)
    vmask = jnp.broadcast_to(
        (ids > 0).astype(jnp.float32)[:, None], (_N, _HW)).reshape(_N * _HW)

    out = pl.pallas_call(
        _bce_body,
        grid=(_NBLK,),
        in_specs=[
            pl.BlockSpec((_FB,), lambda i: (i,)),
            pl.BlockSpec((_FB,), lambda i: (i,)),
            pl.BlockSpec((_FB,), lambda i: (i,)),
        ],
        out_specs=pl.BlockSpec(
            (1, 2), lambda i: (0, 0), memory_space=pltpu.SMEM
        ),
        out_shape=jax.ShapeDtypeStruct((1, 2), jnp.float32),
    )(t.reshape(_N * _HW), yp, vmask)
    return out[0, 0]
